# Initial kernel scaffold; baseline (speedup 1.0000x reference)
import jax
import jax.numpy as jnp
from jax.experimental import pallas as pl


def kernel(x, edge_index, edge_weight, W):
    raise NotImplementedError("write your pallas kernel here")


# speedup vs baseline: 3.8025x; 3.8025x over previous
import functools

import jax
import jax.numpy as jnp
from jax import lax
from jax.experimental import pallas as pl
from jax.experimental.pallas import tpu as pltpu
from jax.experimental.pallas import tpu_sc as plsc

NC = 2
NS = 16
LANES = 16
CHUNK = 128


def _matmul_body(x_ref, w_ref, o_ref):
    o_ref[...] = jnp.dot(x_ref[...], w_ref[...],
                         preferred_element_type=jnp.float32)


def _combine_body(p_ref, o_ref):
    o_ref[...] = jnp.maximum(p_ref[0] + p_ref[1], 0.0)


def _make_sc_scatter(n_pad, d, e_pad, chunks_per_worker):
    rows_per_tile = n_pad // NS
    mesh = plsc.VectorSubcoreMesh(core_axis_name="c", subcore_axis_name="s")

    @functools.partial(
        pl.kernel,
        mesh=mesh,
        out_type=jax.ShapeDtypeStruct((NC, n_pad, d), jnp.float32),
        scratch_types=[
            pltpu.VMEM((CHUNK,), jnp.int32),
            pltpu.VMEM((CHUNK,), jnp.int32),
            pltpu.VMEM((CHUNK,), jnp.float32),
            pltpu.VMEM((CHUNK, d), jnp.float32),
            pltpu.VMEM((CHUNK, d), jnp.float32),
            pltpu.VMEM_SHARED((n_pad, d), jnp.float32),
            pltpu.SemaphoreType.DMA,
        ],
    )
    def sc_scatter(support_hbm, src_hbm, dst_hbm, w_hbm, out_hbm,
                   src_v, dst_v, w_v, rows_v, zero_v, acc_sh, sem):
        cid = lax.axis_index("c")
        sid = lax.axis_index("s")
        wid = cid * NS + sid

        def zero_body(i, _):
            r = i // (d // LANES)
            j = i % (d // LANES)
            zero_v[r, pl.ds(j * LANES, LANES)] = jnp.zeros(
                (LANES,), jnp.float32)
            return 0
        lax.fori_loop(0, CHUNK * (d // LANES), zero_body, 0)

        r0 = sid * rows_per_tile
        off = 0
        while off < rows_per_tile:
            nb = min(CHUNK, rows_per_tile - off)
            pltpu.sync_copy(zero_v.at[pl.ds(0, nb)],
                            acc_sh.at[pl.ds(r0 + off, nb)])
            off += nb
        plsc.subcore_barrier()

        base = wid * (chunks_per_worker * CHUNK)

        def chunk_body(g, _):
            eoff = base + g * CHUNK
            pltpu.sync_copy(src_hbm.at[pl.ds(eoff, CHUNK)], src_v)
            pltpu.sync_copy(dst_hbm.at[pl.ds(eoff, CHUNK)], dst_v)
            pltpu.sync_copy(w_hbm.at[pl.ds(eoff, CHUNK)], w_v)
            pltpu.async_copy(support_hbm.at[src_v], rows_v, sem).wait()

            def scale_body(g, _):
                wv = w_v[pl.ds(g * LANES, LANES)]
                for k in range(LANES):
                    w = wv[k]
                    e = g * LANES + k
                    for j in range(d // LANES):
                        sl = pl.ds(j * LANES, LANES)
                        rows_v[e, sl] = rows_v[e, sl] * w
                return 0
            lax.fori_loop(0, CHUNK // LANES, scale_body, 0)

            pltpu.sync_copy(rows_v, acc_sh.at[dst_v], add=True)
            return 0
        lax.fori_loop(0, chunks_per_worker, chunk_body, 0)

        plsc.subcore_barrier()
        pltpu.sync_copy(acc_sh.at[pl.ds(r0, rows_per_tile)],
                        out_hbm.at[cid, pl.ds(r0, rows_per_tile)])

    return sc_scatter


def kernel(x, edge_index, edge_weight, W):
    n, d_in = x.shape
    d_out = W.shape[1]
    e = edge_weight.shape[0]
    nw = NC * NS

    bm = 1000 if n % 1000 == 0 else n
    support = pl.pallas_call(
        _matmul_body,
        grid=(n // bm,),
        in_specs=[
            pl.BlockSpec((bm, d_in), lambda i: (i, 0)),
            pl.BlockSpec((d_in, d_out), lambda i: (0, 0)),
        ],
        out_specs=pl.BlockSpec((bm, d_out), lambda i: (i, 0)),
        out_shape=jax.ShapeDtypeStruct((n, d_out), jnp.float32),
    )(x, W)

    e_per_worker = -(-e // nw)
    chunks_per_worker = -(-e_per_worker // CHUNK)
    e_pad = nw * chunks_per_worker * CHUNK
    pad = e_pad - e
    src = edge_index[0]
    dst = edge_index[1]
    if pad:
        zi = jnp.zeros((pad,), jnp.int32)
        src = jnp.concatenate([src, zi])
        dst = jnp.concatenate([dst, zi])
        edge_weight = jnp.concatenate(
            [edge_weight, jnp.zeros((pad,), jnp.float32)])

    n_pad = NS * (-(-n // (NS * 8)) * 8)
    sc_scatter = _make_sc_scatter(n_pad, d_out, e_pad, chunks_per_worker)
    partials = sc_scatter(support, src, dst, edge_weight)

    bmc = n_pad // NS
    out_pad = pl.pallas_call(
        _combine_body,
        grid=(n_pad // bmc,),
        in_specs=[pl.BlockSpec((NC, bmc, d_out), lambda i: (0, i, 0))],
        out_specs=pl.BlockSpec((bmc, d_out), lambda i: (i, 0)),
        out_shape=jax.ShapeDtypeStruct((n_pad, d_out), jnp.float32),
    )(partials)
    return out_pad[:n] if n_pad != n else out_pad


# speedup vs baseline: 4.7402x; 1.2466x over previous
import functools

import jax
import jax.numpy as jnp
from jax import lax
from jax.experimental import pallas as pl
from jax.experimental.pallas import tpu as pltpu
from jax.experimental.pallas import tpu_sc as plsc

NC = 2
NS = 16
LANES = 16
CHUNK = 128


def _matmul_body(x_ref, w_ref, o_ref):
    o_ref[...] = jnp.dot(x_ref[...], w_ref[...],
                         preferred_element_type=jnp.float32)


def _combine_body(p_ref, o_ref):
    o_ref[...] = jnp.maximum(p_ref[0] + p_ref[1], 0.0)


def _make_sc_scatter(n_pad, d, g_chunks):
    rows_per_tile = n_pad // NS
    mesh = plsc.VectorSubcoreMesh(core_axis_name="c", subcore_axis_name="s")

    @functools.partial(
        pl.kernel,
        mesh=mesh,
        out_type=jax.ShapeDtypeStruct((NC, n_pad, d), jnp.float32),
        scratch_types=[
            pltpu.VMEM((g_chunks * CHUNK,), jnp.int32),
            pltpu.VMEM((g_chunks, CHUNK), jnp.int32),
            pltpu.VMEM((g_chunks * CHUNK,), jnp.float32),
            pltpu.VMEM((CHUNK, d), jnp.float32),
            pltpu.VMEM_SHARED((n_pad, d), jnp.float32),
            pltpu.SemaphoreType.DMA,
        ],
    )
    def sc_scatter(support_hbm, src_hbm, dst_hbm, w_hbm, out_hbm,
                   src_all, dst_all, w_v, rows_v, acc_sh, sem):
        cid = lax.axis_index("c")
        sid = lax.axis_index("s")
        wid = cid * NS + sid

        def zero_body(i, _):
            r = i // (d // LANES)
            j = i % (d // LANES)
            rows_v[r, pl.ds(j * LANES, LANES)] = jnp.zeros(
                (LANES,), jnp.float32)
            return 0
        lax.fori_loop(0, CHUNK * (d // LANES), zero_body, 0)

        r0 = sid * rows_per_tile
        off = 0
        while off < rows_per_tile:
            nb = min(CHUNK, rows_per_tile - off)
            pltpu.sync_copy(rows_v.at[pl.ds(0, nb)],
                            acc_sh.at[pl.ds(r0 + off, nb)])
            off += nb
        plsc.subcore_barrier()

        base = wid * (g_chunks * CHUNK)
        pltpu.sync_copy(w_hbm.at[pl.ds(base, g_chunks * CHUNK)], w_v)
        pltpu.sync_copy(src_hbm.at[pl.ds(base, g_chunks * CHUNK)], src_all)
        pltpu.sync_copy(dst_hbm.at[wid], dst_all)

        def chunk_body(g, _):
            pltpu.async_copy(
                support_hbm.at[src_all.at[pl.ds(g * CHUNK, CHUNK)]],
                rows_v, sem).wait()

            def scale_body(k, _):
                wv = w_v[pl.ds(g * CHUNK + k * LANES, LANES)]
                for kk in range(LANES):
                    w = wv[kk]
                    e = k * LANES + kk
                    for j in range(d // LANES):
                        sl = pl.ds(j * LANES, LANES)
                        rows_v[e, sl] = rows_v[e, sl] * w
                return 0
            lax.fori_loop(0, CHUNK // LANES, scale_body, 0)

            pltpu.sync_copy(rows_v, acc_sh.at[dst_all.at[g]], add=True)
            return 0
        lax.fori_loop(0, g_chunks, chunk_body, 0)

        plsc.subcore_barrier()
        pltpu.sync_copy(acc_sh.at[pl.ds(r0, rows_per_tile)],
                        out_hbm.at[cid, pl.ds(r0, rows_per_tile)])

    return sc_scatter


def kernel(x, edge_index, edge_weight, W):
    n, d_in = x.shape
    d_out = W.shape[1]
    e = edge_weight.shape[0]
    nw = NC * NS

    bm = 1000 if n % 1000 == 0 else n
    support = pl.pallas_call(
        _matmul_body,
        grid=(n // bm,),
        in_specs=[
            pl.BlockSpec((bm, d_in), lambda i: (i, 0)),
            pl.BlockSpec((d_in, d_out), lambda i: (0, 0)),
        ],
        out_specs=pl.BlockSpec((bm, d_out), lambda i: (i, 0)),
        out_shape=jax.ShapeDtypeStruct((n, d_out), jnp.float32),
    )(x, W)

    e_per_worker = -(-e // nw)
    g_chunks = -(-e_per_worker // CHUNK)
    e_pad = nw * g_chunks * CHUNK
    pad = e_pad - e
    src = edge_index[0]
    dst = edge_index[1]
    if pad:
        zi = jnp.zeros((pad,), jnp.int32)
        src = jnp.concatenate([src, zi])
        dst = jnp.concatenate([dst, zi])
        edge_weight = jnp.concatenate(
            [edge_weight, jnp.zeros((pad,), jnp.float32)])
    dst = dst.reshape(nw, g_chunks, CHUNK)

    n_pad = NS * (-(-n // (NS * 8)) * 8)
    sc_scatter = _make_sc_scatter(n_pad, d_out, g_chunks)
    partials = sc_scatter(support, src, dst, edge_weight)

    bmc = n_pad // NS
    out_pad = pl.pallas_call(
        _combine_body,
        grid=(n_pad // bmc,),
        in_specs=[pl.BlockSpec((NC, bmc, d_out), lambda i: (0, i, 0))],
        out_specs=pl.BlockSpec((bmc, d_out), lambda i: (i, 0)),
        out_shape=jax.ShapeDtypeStruct((n_pad, d_out), jnp.float32),
    )(partials)
    return out_pad[:n] if n_pad != n else out_pad
